# Initial kernel scaffold; baseline (speedup 1.0000x reference)
import jax
import jax.numpy as jnp
from jax.experimental import pallas as pl


def kernel(embed, w0, w2, w3, drop_scale):
    raise NotImplementedError("write your pallas kernel here")


# speedup vs baseline: 1.6820x; 1.6820x over previous
from functools import partial

import jax
import jax.numpy as jnp
from jax.experimental import pallas as pl
from jax.experimental.pallas import tpu as pltpu


def _sign(x):
    return jnp.where(x > 0, 1.0, jnp.where(x < 0, -1.0, 0.0))


def _fc2_kernel(x_ref, w2t_ref, drop_ref, h_ref):
    w2t = w2t_ref[...]
    inv2 = jax.lax.rsqrt(jnp.sum(w2t * w2t, axis=0, keepdims=True) + 1e-6)
    h = jnp.dot(x_ref[...], w2t, preferred_element_type=jnp.float32)
    h_ref[...] = h * inv2 * drop_ref[...]


def _fc3_fc0_kernel(h_ref, w3t_ref, x0_ref, w0t_ref, o_ref, b_ref, *, n_i):
    b = jnp.dot(x0_ref[0], w0t_ref[...], preferred_element_type=jnp.float32)
    b_ref[...] = _sign(b)[None]

    w3t = w3t_ref[...]
    c3 = w3t.shape[1]
    m = h_ref.shape[0] // n_i
    for j in range(n_i):
        p = jnp.dot(h_ref[j * m:(j + 1) * m, :], w3t,
                    preferred_element_type=jnp.float32)
        o_ref[:, j * c3:(j + 1) * c3] = _sign(p)


def kernel(embed, w0, w2, w3, drop_scale):
    N_O, N_I, E = embed.shape
    Cout = w0.shape[0]
    C3 = w3.shape[0]
    C2 = w2.shape[0]
    N = N_O * N_I
    f32 = jnp.float32

    x = embed.reshape(N, E).astype(f32)
    x0 = embed.reshape(N_O, 1, N_I * E).astype(f32)
    w0_t = jnp.transpose(w0).astype(f32)
    w2_t = jnp.transpose(w2).astype(f32)
    w3_t = jnp.transpose(w3).astype(f32)

    row_blk = 128 if N % 128 == 0 else N
    h = pl.pallas_call(
        _fc2_kernel,
        out_shape=jax.ShapeDtypeStruct((N, C2), f32),
        grid=(N // row_blk,),
        in_specs=[
            pl.BlockSpec((row_blk, E), lambda i: (i, 0)),
            pl.BlockSpec((E, C2), lambda i: (0, 0)),
            pl.BlockSpec((row_blk, C2), lambda i: (i, 0)),
        ],
        out_specs=pl.BlockSpec((row_blk, C2), lambda i: (i, 0)),
        compiler_params=pltpu.CompilerParams(
            dimension_semantics=("parallel",)),
    )(x, w2_t, drop_scale)

    h2 = h.reshape(N * Cout, E)

    o, b = pl.pallas_call(
        partial(_fc3_fc0_kernel, n_i=N_I),
        out_shape=(jax.ShapeDtypeStruct((N_O * Cout, N_I * C3), f32),
                   jax.ShapeDtypeStruct((N_O, 1, Cout), f32)),
        grid=(N_O,),
        in_specs=[
            pl.BlockSpec((N_I * Cout, E), lambda i: (i, 0)),
            pl.BlockSpec((E, C3), lambda i: (0, 0)),
            pl.BlockSpec((1, 1, N_I * E), lambda i: (i, 0, 0)),
            pl.BlockSpec((N_I * E, Cout), lambda i: (0, 0)),
        ],
        out_specs=(pl.BlockSpec((Cout, N_I * C3), lambda i: (i, 0)),
                   pl.BlockSpec((1, 1, Cout), lambda i: (i, 0, 0))),
        compiler_params=pltpu.CompilerParams(
            dimension_semantics=("parallel",)),
    )(h2, w3_t, x0, w0_t)

    K = 3
    Cin = C3 // (K * K)
    weight = o.reshape(N_O * Cout, N_I * Cin, K, K)
    bias = b.reshape(N_O * Cout, 1)
    return weight, bias


# speedup vs baseline: 1.7932x; 1.0661x over previous
from functools import partial

import jax
import jax.numpy as jnp
from jax.experimental import pallas as pl
from jax.experimental.pallas import tpu as pltpu


def _sign(x):
    return jnp.where(x > 0, 1.0, jnp.where(x < 0, -1.0, 0.0))


def _fused_kernel(x_ref, w2t_ref, drop_ref, w3t_ref, x0_ref, w0t_ref,
                  o_ref, b_ref, *, n_i):
    b = jnp.dot(x0_ref[0], w0t_ref[...], preferred_element_type=jnp.float32)
    b_ref[...] = _sign(b)[None]

    w2t = w2t_ref[...]
    inv2 = jax.lax.rsqrt(jnp.sum(w2t * w2t, axis=0, keepdims=True) + 1e-6)
    h16 = jnp.dot(x_ref[...], w2t, preferred_element_type=jnp.float32)
    h16 = h16 * inv2 * drop_ref[...]

    e = x_ref.shape[1]
    cout = h16.shape[1] // e
    hm = jnp.stack([h16[:, c * e:(c + 1) * e] for c in range(cout)],
                   axis=1).reshape(n_i * cout, e)

    w3t = w3t_ref[...]
    c3 = w3t.shape[1]
    p = _sign(jnp.dot(hm, w3t, preferred_element_type=jnp.float32))
    for j in range(n_i):
        o_ref[:, j * c3:(j + 1) * c3] = p[j * cout:(j + 1) * cout, :]


def kernel(embed, w0, w2, w3, drop_scale):
    N_O, N_I, E = embed.shape
    Cout = w0.shape[0]
    C3 = w3.shape[0]
    C2 = w2.shape[0]
    N = N_O * N_I
    f32 = jnp.float32

    x = embed.reshape(N, E).astype(f32)
    x0 = embed.reshape(N_O, 1, N_I * E).astype(f32)
    w0_t = jnp.transpose(w0).astype(f32)
    w2_t = jnp.transpose(w2).astype(f32)
    w3_t = jnp.transpose(w3).astype(f32)

    o, b = pl.pallas_call(
        partial(_fused_kernel, n_i=N_I),
        out_shape=(jax.ShapeDtypeStruct((N_O * Cout, N_I * C3), f32),
                   jax.ShapeDtypeStruct((N_O, 1, Cout), f32)),
        grid=(N_O,),
        in_specs=[
            pl.BlockSpec((N_I, E), lambda i: (i, 0)),
            pl.BlockSpec((E, C2), lambda i: (0, 0)),
            pl.BlockSpec((N_I, C2), lambda i: (i, 0)),
            pl.BlockSpec((E, C3), lambda i: (0, 0)),
            pl.BlockSpec((1, 1, N_I * E), lambda i: (i, 0, 0)),
            pl.BlockSpec((N_I * E, Cout), lambda i: (0, 0)),
        ],
        out_specs=(pl.BlockSpec((Cout, N_I * C3), lambda i: (i, 0)),
                   pl.BlockSpec((1, 1, Cout), lambda i: (i, 0, 0))),
        compiler_params=pltpu.CompilerParams(
            dimension_semantics=("parallel",)),
    )(x, w2_t, drop_scale, w3_t, x0, w0_t)

    K = 3
    Cin = C3 // (K * K)
    weight = o.reshape(N_O * Cout, N_I * Cin, K, K)
    bias = b.reshape(N_O * Cout, 1)
    return weight, bias


# speedup vs baseline: 8.2787x; 4.6166x over previous
from functools import partial

import jax
import jax.numpy as jnp
from jax.experimental import pallas as pl
from jax.experimental.pallas import tpu as pltpu


def _sign(x):
    return jnp.where(x > 0, 1.0, jnp.where(x < 0, -1.0, 0.0))


def _fused_kernel(x_ref, w2t_ref, drop_ref, w3r_ref, x0_ref, w0t_ref,
                  o_ref, b_ref, *, n_i, kk):
    b = jnp.dot(x0_ref[0], w0t_ref[...], preferred_element_type=jnp.float32)
    b_ref[...] = _sign(b)[None]

    w2t = w2t_ref[...]
    inv2 = jax.lax.rsqrt(jnp.sum(w2t * w2t, axis=0, keepdims=True) + 1e-6)
    h16 = jnp.dot(x_ref[...], w2t, preferred_element_type=jnp.float32)
    h16 = h16 * inv2 * drop_ref[...]

    e = x_ref.shape[1]
    cout = h16.shape[1] // e
    hm = jnp.stack([h16[:, c * e:(c + 1) * e] for c in range(cout)],
                   axis=1).reshape(n_i * cout, e)

    c3 = w3r_ref.shape[1]
    cin = c3 // kk
    p = _sign(jnp.dot(hm, w3r_ref[...], preferred_element_type=jnp.float32))

    for q in range(kk):
        o_ref[q] = jnp.concatenate(
            [p[j * cout:(j + 1) * cout, q * cin:(q + 1) * cin]
             for j in range(n_i)], axis=1)


def kernel(embed, w0, w2, w3, drop_scale):
    N_O, N_I, E = embed.shape
    Cout = w0.shape[0]
    C3 = w3.shape[0]
    C2 = w2.shape[0]
    N = N_O * N_I
    K = 3
    KK = K * K
    Cin = C3 // KK
    f32 = jnp.float32

    x = embed.reshape(N, E).astype(f32)
    x0 = embed.reshape(N_O, 1, N_I * E).astype(f32)
    w0_t = jnp.transpose(w0).astype(f32)
    w2_t = jnp.transpose(w2).astype(f32)
    w3_r = jnp.transpose(w3).astype(f32).reshape(E, Cin, KK)
    w3_r = jnp.transpose(w3_r, (0, 2, 1)).reshape(E, C3)

    o9, b = pl.pallas_call(
        partial(_fused_kernel, n_i=N_I, kk=KK),
        out_shape=(jax.ShapeDtypeStruct((KK, N_O * Cout, N_I * Cin), f32),
                   jax.ShapeDtypeStruct((N_O, 1, Cout), f32)),
        grid=(N_O,),
        in_specs=[
            pl.BlockSpec((N_I, E), lambda i: (i, 0)),
            pl.BlockSpec((E, C2), lambda i: (0, 0)),
            pl.BlockSpec((N_I, C2), lambda i: (i, 0)),
            pl.BlockSpec((E, C3), lambda i: (0, 0)),
            pl.BlockSpec((1, 1, N_I * E), lambda i: (i, 0, 0)),
            pl.BlockSpec((N_I * E, Cout), lambda i: (0, 0)),
        ],
        out_specs=(pl.BlockSpec((KK, Cout, N_I * Cin), lambda i: (0, i, 0)),
                   pl.BlockSpec((1, 1, Cout), lambda i: (i, 0, 0))),
        compiler_params=pltpu.CompilerParams(
            dimension_semantics=("parallel",)),
    )(x, w2_t, drop_scale, w3_r, x0, w0_t)

    weight = jnp.transpose(o9.reshape(K, K, N_O * Cout, N_I * Cin),
                           (2, 3, 0, 1))
    bias = b.reshape(N_O * Cout, 1)
    return weight, bias


# speedup vs baseline: 8.4210x; 1.0172x over previous
from functools import partial

import jax
import jax.numpy as jnp
from jax.experimental import pallas as pl
from jax.experimental.pallas import tpu as pltpu


def _sign(x):
    return jnp.where(x > 0, 1.0, jnp.where(x < 0, -1.0, 0.0))


def _fused_kernel(x_ref, w2s_ref, drop_ref, w3r_ref, w0_ref,
                  o_ref, b_ref, *, n_i, kk):
    e = x_ref.shape[1]
    b = jnp.zeros((1, w0_ref.shape[0]), jnp.float32)
    for j in range(n_i):
        b = b + jax.lax.dot_general(
            x_ref[j:j + 1, :], w0_ref[:, j * e:(j + 1) * e],
            (((1,), (1,)), ((), ())), preferred_element_type=jnp.float32)
    b_ref[...] = _sign(b)[None]

    h16 = jnp.dot(x_ref[...], w2s_ref[...], preferred_element_type=jnp.float32)
    h16 = h16 * drop_ref[...]

    cout = h16.shape[1] // e
    hm = jnp.stack([h16[:, c * e:(c + 1) * e] for c in range(cout)],
                   axis=1).reshape(n_i * cout, e)

    c3 = w3r_ref.shape[1]
    cin = c3 // kk
    p = _sign(jnp.dot(hm, w3r_ref[...], preferred_element_type=jnp.float32))

    for q in range(kk):
        o_ref[q] = jnp.concatenate(
            [p[j * cout:(j + 1) * cout, q * cin:(q + 1) * cin]
             for j in range(n_i)], axis=1)


def kernel(embed, w0, w2, w3, drop_scale):
    N_O, N_I, E = embed.shape
    Cout = w0.shape[0]
    C3 = w3.shape[0]
    C2 = w2.shape[0]
    N = N_O * N_I
    K = 3
    KK = K * K
    Cin = C3 // KK
    f32 = jnp.float32

    x = embed.reshape(N, E).astype(f32)
    w2_t = jnp.transpose(w2).astype(f32)
    w2_s = w2_t * jax.lax.rsqrt(
        jnp.sum(w2_t * w2_t, axis=0, keepdims=True) + 1e-6)
    w3_r = jnp.transpose(w3).astype(f32).reshape(E, Cin, KK)
    w3_r = jnp.transpose(w3_r, (0, 2, 1)).reshape(E, C3)

    o9, b = pl.pallas_call(
        partial(_fused_kernel, n_i=N_I, kk=KK),
        out_shape=(jax.ShapeDtypeStruct((KK, N_O * Cout, N_I * Cin), f32),
                   jax.ShapeDtypeStruct((N_O, 1, Cout), f32)),
        grid=(N_O,),
        in_specs=[
            pl.BlockSpec((N_I, E), lambda i: (i, 0)),
            pl.BlockSpec((E, C2), lambda i: (0, 0)),
            pl.BlockSpec((N_I, C2), lambda i: (i, 0)),
            pl.BlockSpec((E, C3), lambda i: (0, 0)),
            pl.BlockSpec((Cout, N_I * E), lambda i: (0, 0)),
        ],
        out_specs=(pl.BlockSpec((KK, Cout, N_I * Cin), lambda i: (0, i, 0)),
                   pl.BlockSpec((1, 1, Cout), lambda i: (i, 0, 0))),
        compiler_params=pltpu.CompilerParams(
            dimension_semantics=("parallel",)),
    )(x, w2_s, drop_scale, w3_r, w0.astype(f32))

    weight = jnp.transpose(o9.reshape(K, K, N_O * Cout, N_I * Cin),
                           (2, 3, 0, 1))
    bias = b.reshape(N_O * Cout, 1)
    return weight, bias


# speedup vs baseline: 8.5396x; 1.0141x over previous
from functools import partial

import jax
import jax.numpy as jnp
from jax.experimental import pallas as pl
from jax.experimental.pallas import tpu as pltpu


def _sign(x):
    return jnp.where(x > 0, 1.0, jnp.where(x < 0, -1.0, 0.0))


def _inv_norm_kernel(w2t_ref, inv_ref):
    w2t = w2t_ref[...]
    inv_ref[...] = jax.lax.rsqrt(
        jnp.sum(w2t * w2t, axis=0, keepdims=True) + 1e-6)


def _fused_kernel(x_ref, w2t_ref, inv2_ref, drop_ref, w3r_ref, w0_ref,
                  o_ref, b_ref, *, n_i, kk):
    e = x_ref.shape[1]
    b = jnp.zeros((1, w0_ref.shape[0]), jnp.float32)
    for j in range(n_i):
        b = b + jax.lax.dot_general(
            x_ref[j:j + 1, :], w0_ref[:, j * e:(j + 1) * e],
            (((1,), (1,)), ((), ())), preferred_element_type=jnp.float32)
    b_ref[...] = _sign(b)[None]

    h16 = jnp.dot(x_ref[...], w2t_ref[...], preferred_element_type=jnp.float32)
    h16 = h16 * inv2_ref[...] * drop_ref[...]

    cout = h16.shape[1] // e
    hm = jnp.stack([h16[:, c * e:(c + 1) * e] for c in range(cout)],
                   axis=1).reshape(n_i * cout, e)

    c3 = w3r_ref.shape[1]
    cin = c3 // kk
    p = _sign(jnp.dot(hm, w3r_ref[...], preferred_element_type=jnp.float32))

    for q in range(kk):
        o_ref[q] = jnp.concatenate(
            [p[j * cout:(j + 1) * cout, q * cin:(q + 1) * cin]
             for j in range(n_i)], axis=1)


def kernel(embed, w0, w2, w3, drop_scale):
    N_O, N_I, E = embed.shape
    Cout = w0.shape[0]
    C3 = w3.shape[0]
    C2 = w2.shape[0]
    N = N_O * N_I
    K = 3
    KK = K * K
    Cin = C3 // KK
    f32 = jnp.float32

    x = embed.reshape(N, E).astype(f32)
    w2_t = jnp.transpose(w2).astype(f32)
    w3_r = jnp.transpose(w3).astype(f32).reshape(E, Cin, KK)
    w3_r = jnp.transpose(w3_r, (0, 2, 1)).reshape(E, C3)

    inv2 = pl.pallas_call(
        _inv_norm_kernel,
        out_shape=jax.ShapeDtypeStruct((1, C2), f32),
        grid=(1,),
        in_specs=[pl.BlockSpec((E, C2), lambda i: (0, 0))],
        out_specs=pl.BlockSpec((1, C2), lambda i: (0, 0)),
        compiler_params=pltpu.CompilerParams(
            dimension_semantics=("arbitrary",)),
    )(w2_t)

    o9, b = pl.pallas_call(
        partial(_fused_kernel, n_i=N_I, kk=KK),
        out_shape=(jax.ShapeDtypeStruct((KK, N_O * Cout, N_I * Cin), f32),
                   jax.ShapeDtypeStruct((N_O, 1, Cout), f32)),
        grid=(N_O,),
        in_specs=[
            pl.BlockSpec((N_I, E), lambda i: (i, 0)),
            pl.BlockSpec((E, C2), lambda i: (0, 0)),
            pl.BlockSpec((1, C2), lambda i: (0, 0)),
            pl.BlockSpec((N_I, C2), lambda i: (i, 0)),
            pl.BlockSpec((E, C3), lambda i: (0, 0)),
            pl.BlockSpec((Cout, N_I * E), lambda i: (0, 0)),
        ],
        out_specs=(pl.BlockSpec((KK, Cout, N_I * Cin), lambda i: (0, i, 0)),
                   pl.BlockSpec((1, 1, Cout), lambda i: (i, 0, 0))),
        compiler_params=pltpu.CompilerParams(
            dimension_semantics=("parallel",)),
    )(x, w2_t, inv2, drop_scale, w3_r, w0.astype(f32))

    weight = jnp.transpose(o9.reshape(K, K, N_O * Cout, N_I * Cin),
                           (2, 3, 0, 1))
    bias = b.reshape(N_O * Cout, 1)
    return weight, bias
